# baseline (speedup 1.0000x reference)
import functools

import jax
import jax.numpy as jnp
from jax import lax
from jax.experimental import pallas as pl
from jax.experimental.pallas import tpu as pltpu
from jax.experimental.pallas import tpu_sc as plsc

N_SEGMENTS = 10000
N_ROWS = 320000
DIM = 128

NUM_CORES = 2
NUM_SUBCORES = 16
NUM_TILES = NUM_CORES * NUM_SUBCORES
ROWS_PER_TILE = N_ROWS // NUM_TILES
CHUNK = 128
NUM_CHUNKS = ROWS_PER_TILE // CHUNK
TAIL = ROWS_PER_TILE - NUM_CHUNKS * CHUNK
NBUF = 3
SEG_PER_TILE = 624
SEG_TAIL = N_SEGMENTS - NUM_SUBCORES * SEG_PER_TILE
ZROWS = 8
NZCOPY = SEG_PER_TILE // ZROWS


def _sc_partials(v, batch):
    mesh = plsc.VectorSubcoreMesh(core_axis_name="c", subcore_axis_name="s")

    @functools.partial(
        pl.kernel,
        out_type=jax.ShapeDtypeStruct((NUM_CORES * N_SEGMENTS, DIM), jnp.float32),
        mesh=mesh,
        scratch_types=[
            pltpu.VMEM((NBUF, CHUNK, DIM), jnp.float32),
            pltpu.VMEM((NBUF, CHUNK), jnp.int32),
            pltpu.VMEM((1, TAIL), jnp.int32),
            pltpu.VMEM((ZROWS, DIM), jnp.float32),
            pltpu.VMEM_SHARED((N_SEGMENTS, DIM), jnp.float32),
            pltpu.SemaphoreType.DMA((NBUF,)),
            pltpu.SemaphoreType.DMA((NBUF,)),
            pltpu.SemaphoreType.DMA,
        ],
    )
    def sc_kernel(v_hbm, batch_hbm, out_hbm, vbuf, idxbuf, tidx, zbuf, acc,
                  lsem, ssem, zsem):
        c = lax.axis_index("c")
        s = lax.axis_index("s")
        tile = c * NUM_SUBCORES + s
        row_base = tile * ROWS_PER_TILE

        def start_load(k, b):
            r0 = row_base + k * CHUNK
            pltpu.async_copy(
                batch_hbm.at[pl.ds(r0, CHUNK)], idxbuf.at[b], lsem.at[b])
            pltpu.async_copy(v_hbm.at[pl.ds(r0, CHUNK)], vbuf.at[b], lsem.at[b])

        def wait_load(b):
            pltpu.make_async_copy(
                batch_hbm.at[pl.ds(0, CHUNK)], idxbuf.at[b], lsem.at[b]).wait()
            pltpu.make_async_copy(
                v_hbm.at[pl.ds(0, CHUNK)], vbuf.at[b], lsem.at[b]).wait()

        def start_scatter(b):
            pltpu.async_copy(
                vbuf.at[b], acc.at[idxbuf.at[b]], ssem.at[b], add=True)

        def wait_scatter(b):
            pltpu.make_async_copy(
                vbuf.at[b], acc.at[idxbuf.at[b]], ssem.at[b]).wait()

        start_load(0, 0)
        start_load(1, 1)

        @pl.loop(0, ZROWS)
        def _(i):
            @pl.loop(0, DIM, step=16)
            def _(j):
                zbuf[pl.ds(i, 1), pl.ds(j, 16)] = jnp.zeros((1, 16), jnp.float32)

        for z in range(NZCOPY):
            pltpu.async_copy(
                zbuf, acc.at[pl.ds(s * SEG_PER_TILE + z * ZROWS, ZROWS)], zsem)

        @pl.when(s == NUM_SUBCORES - 1)
        def _():
            for t in range(SEG_TAIL // ZROWS):
                pltpu.async_copy(
                    zbuf,
                    acc.at[
                        pl.ds(NUM_SUBCORES * SEG_PER_TILE + t * ZROWS, ZROWS)],
                    zsem,
                )

        for z in range(NZCOPY):
            pltpu.make_async_copy(
                zbuf, acc.at[pl.ds(0, ZROWS)], zsem).wait()

        @pl.when(s == NUM_SUBCORES - 1)
        def _():
            for t in range(SEG_TAIL // ZROWS):
                pltpu.make_async_copy(
                    zbuf, acc.at[pl.ds(0, ZROWS)], zsem).wait()

        plsc.subcore_barrier()

        @pl.loop(0, NUM_CHUNKS // NBUF)
        def _(j):
            for b in range(NBUF):
                k = j * NBUF + b

                if b >= NBUF - 2:
                    wait_scatter((b + 2) % NBUF)
                else:
                    @pl.when(j > 0)
                    def _():
                        wait_scatter((b + 2) % NBUF)

                @pl.when(k + 2 < NUM_CHUNKS)
                def _():
                    start_load(k + 2, (b + 2) % NBUF)

                wait_load(b)
                start_scatter(b)

        wait_scatter((NUM_CHUNKS - 1) % NBUF)

        r0 = row_base + NUM_CHUNKS * CHUNK
        pltpu.sync_copy(batch_hbm.at[pl.ds(r0, TAIL)], tidx.at[0])
        pltpu.sync_copy(v_hbm.at[pl.ds(r0, TAIL)],
                        vbuf.at[0, pl.ds(0, TAIL)])
        pltpu.sync_copy(vbuf.at[0, pl.ds(0, TAIL)],
                        acc.at[tidx.at[0]], add=True)

        plsc.subcore_barrier()

        out_base = c * N_SEGMENTS + s * SEG_PER_TILE
        pltpu.sync_copy(
            acc.at[pl.ds(s * SEG_PER_TILE, SEG_PER_TILE)],
            out_hbm.at[pl.ds(out_base, SEG_PER_TILE)],
        )

        @pl.when(s == NUM_SUBCORES - 1)
        def _():
            pltpu.sync_copy(
                acc.at[pl.ds(NUM_SUBCORES * SEG_PER_TILE, SEG_TAIL)],
                out_hbm.at[
                    pl.ds(c * N_SEGMENTS + NUM_SUBCORES * SEG_PER_TILE, SEG_TAIL)
                ],
            )

    return sc_kernel(v, batch)


def _combine(u, partials):
    blk = 1000
    nblk = N_SEGMENTS // blk

    def body(u_ref, p0_ref, p1_ref, o_ref):
        o_ref[...] = u_ref[...] + p0_ref[...] + p1_ref[...]

    return pl.pallas_call(
        body,
        grid=(nblk,),
        in_specs=[
            pl.BlockSpec((blk, DIM), lambda i: (i, 0)),
            pl.BlockSpec((blk, DIM), lambda i: (i, 0)),
            pl.BlockSpec((blk, DIM), lambda i: (i + nblk, 0)),
        ],
        out_specs=pl.BlockSpec((blk, DIM), lambda i: (i, 0)),
        out_shape=jax.ShapeDtypeStruct((N_SEGMENTS, DIM), jnp.float32),
    )(u, partials, partials)


def kernel(u, v, batch):
    batch32 = batch.astype(jnp.int32)
    partials = _sc_partials(v, batch32)
    return _combine(u, partials)


# speedup vs baseline: 1.0192x; 1.0192x over previous
import functools

import jax
import jax.numpy as jnp
from jax import lax
from jax.experimental import pallas as pl
from jax.experimental.pallas import tpu as pltpu
from jax.experimental.pallas import tpu_sc as plsc

N_SEGMENTS = 10000
N_ROWS = 320000
DIM = 128

NUM_CORES = 2
NUM_SUBCORES = 16
NUM_TILES = NUM_CORES * NUM_SUBCORES
ROWS_PER_TILE = N_ROWS // NUM_TILES
CHUNK = 80
NUM_CHUNKS = ROWS_PER_TILE // CHUNK
TAIL = ROWS_PER_TILE - NUM_CHUNKS * CHUNK
NBUF = 4
NFULL = (NUM_CHUNKS // NBUF) * NBUF
SEG_PER_TILE = 624
SEG_TAIL = N_SEGMENTS - NUM_SUBCORES * SEG_PER_TILE
ZROWS = 16
NZCOPY = SEG_PER_TILE // ZROWS


def _sc_partials(v, batch):
    mesh = plsc.VectorSubcoreMesh(core_axis_name="c", subcore_axis_name="s")

    @functools.partial(
        pl.kernel,
        out_type=jax.ShapeDtypeStruct((NUM_CORES * N_SEGMENTS, DIM), jnp.float32),
        mesh=mesh,
        scratch_types=[
            pltpu.VMEM((NBUF, CHUNK, DIM), jnp.float32),
            pltpu.VMEM((NBUF, CHUNK), jnp.int32),
            pltpu.VMEM((ZROWS, DIM), jnp.float32),
            pltpu.VMEM_SHARED((N_SEGMENTS, DIM), jnp.float32),
            pltpu.SemaphoreType.DMA((NBUF,)),
            pltpu.SemaphoreType.DMA((NBUF,)),
            pltpu.SemaphoreType.DMA,
        ],
    )
    def sc_kernel(v_hbm, batch_hbm, out_hbm, vbuf, idxbuf, zbuf, acc,
                  lsem, ssem, zsem):
        c = lax.axis_index("c")
        s = lax.axis_index("s")
        tile = c * NUM_SUBCORES + s
        row_base = tile * ROWS_PER_TILE

        def start_load(k, b):
            r0 = row_base + k * CHUNK
            pltpu.async_copy(
                batch_hbm.at[pl.ds(r0, CHUNK)], idxbuf.at[b], lsem.at[b])
            pltpu.async_copy(v_hbm.at[pl.ds(r0, CHUNK)], vbuf.at[b], lsem.at[b])

        def wait_load(b):
            pltpu.make_async_copy(
                batch_hbm.at[pl.ds(0, CHUNK)], idxbuf.at[b], lsem.at[b]).wait()
            pltpu.make_async_copy(
                v_hbm.at[pl.ds(0, CHUNK)], vbuf.at[b], lsem.at[b]).wait()

        def start_scatter(b):
            pltpu.sync_copy(vbuf.at[b], acc.at[idxbuf.at[b]], add=True)

        def wait_scatter(b):
            del b

        start_load(0, 0)
        start_load(1, 1)

        @pl.loop(0, ZROWS)
        def _(i):
            @pl.loop(0, DIM, step=16)
            def _(j):
                zbuf[pl.ds(i, 1), pl.ds(j, 16)] = jnp.zeros((1, 16), jnp.float32)

        for z in range(NZCOPY):
            pltpu.async_copy(
                zbuf, acc.at[pl.ds(s * SEG_PER_TILE + z * ZROWS, ZROWS)], zsem)

        @pl.when(s == NUM_SUBCORES - 1)
        def _():
            pltpu.async_copy(
                zbuf.at[pl.ds(0, SEG_TAIL)],
                acc.at[pl.ds(NUM_SUBCORES * SEG_PER_TILE, SEG_TAIL)],
                zsem,
            )

        for z in range(NZCOPY):
            pltpu.make_async_copy(
                zbuf, acc.at[pl.ds(0, ZROWS)], zsem).wait()

        @pl.when(s == NUM_SUBCORES - 1)
        def _():
            pltpu.make_async_copy(
                zbuf.at[pl.ds(0, SEG_TAIL)],
                acc.at[pl.ds(0, SEG_TAIL)],
                zsem,
            ).wait()

        plsc.subcore_barrier()

        @pl.loop(0, NFULL // NBUF)
        def _(j):
            for b in range(NBUF):
                k = j * NBUF + b

                if b >= 2:
                    wait_scatter((b + 2) % NBUF)
                else:
                    @pl.when(j > 0)
                    def _():
                        wait_scatter((b + 2) % NBUF)

                @pl.when(k + 2 < NUM_CHUNKS)
                def _():
                    start_load(k + 2, (b + 2) % NBUF)

                wait_load(b)
                start_scatter(b)

        wait_scatter((NFULL - 2) % NBUF)
        wait_scatter((NFULL - 1) % NBUF)
        wait_load((NUM_CHUNKS - 1) % NBUF)
        start_scatter((NUM_CHUNKS - 1) % NBUF)
        wait_scatter((NUM_CHUNKS - 1) % NBUF)

        plsc.subcore_barrier()

        out_base = c * N_SEGMENTS + s * SEG_PER_TILE
        pltpu.sync_copy(
            acc.at[pl.ds(s * SEG_PER_TILE, SEG_PER_TILE)],
            out_hbm.at[pl.ds(out_base, SEG_PER_TILE)],
        )

        @pl.when(s == NUM_SUBCORES - 1)
        def _():
            pltpu.sync_copy(
                acc.at[pl.ds(NUM_SUBCORES * SEG_PER_TILE, SEG_TAIL)],
                out_hbm.at[
                    pl.ds(c * N_SEGMENTS + NUM_SUBCORES * SEG_PER_TILE, SEG_TAIL)
                ],
            )

    return sc_kernel(v, batch)


def _combine(u, partials):
    blk = 1000
    nblk = N_SEGMENTS // blk

    def body(u_ref, p0_ref, p1_ref, o_ref):
        o_ref[...] = u_ref[...] + p0_ref[...] + p1_ref[...]

    return pl.pallas_call(
        body,
        grid=(nblk,),
        in_specs=[
            pl.BlockSpec((blk, DIM), lambda i: (i, 0)),
            pl.BlockSpec((blk, DIM), lambda i: (i, 0)),
            pl.BlockSpec((blk, DIM), lambda i: (i + nblk, 0)),
        ],
        out_specs=pl.BlockSpec((blk, DIM), lambda i: (i, 0)),
        out_shape=jax.ShapeDtypeStruct((N_SEGMENTS, DIM), jnp.float32),
    )(u, partials, partials)


def kernel(u, v, batch):
    batch32 = batch.astype(jnp.int32)
    partials = _sc_partials(v, batch32)
    return _combine(u, partials)
